# Initial kernel scaffold; baseline (speedup 1.0000x reference)
import jax
import jax.numpy as jnp
from jax.experimental import pallas as pl


def kernel(h, edge_index, W1, b1, W2, b2, W3, b3):
    raise NotImplementedError("write your pallas kernel here")


# speedup vs baseline: 2.1802x; 2.1802x over previous
import functools

import jax
import jax.numpy as jnp
from jax import lax
from jax.experimental import pallas as pl
from jax.experimental.pallas import tpu as pltpu
from jax.experimental.pallas import tpu_sc as plsc

N = 10000
NP = 10240
E = 320000
D = 128
NW = 32
FPT = D // NW
CH = 4000
NCH = E // CH
SHARD = E // NW
BN = 1280
GRID = NP // BN
VPC = CH // 16
LANES = 16


def _wid():
    return lax.axis_index("s") * 2 + lax.axis_index("c")


def _scatter_add_resolved(acc_ref, idx, val, claim_ref, lane):
    plsc.store_scatter(claim_ref, (idx,), lane)
    won = plsc.load_gather(claim_ref, (idx,))
    safe = won == lane
    plsc.addupdate_scatter(acc_ref, (idx,), val, mask=safe)
    rem = jnp.logical_not(safe)

    def cond(r):
        return jnp.any(r)

    def tail(r):
        plsc.store_scatter(claim_ref, (idx,), lane, mask=r)
        w2 = plsc.load_gather(claim_ref, (idx,), mask=r)
        ok = jnp.logical_and(r, w2 == lane)
        plsc.addupdate_scatter(acc_ref, (idx,), val, mask=ok)
        return jnp.logical_and(r, jnp.logical_not(ok))

    lax.while_loop(cond, tail, rem)


def _zero_ref(ref, nwords):
    z = jnp.zeros((LANES,), jnp.float32)

    def body(i, _):
        ref[pl.ds(i * LANES, LANES)] = z
        return 0

    lax.fori_loop(0, nwords // LANES, body, 0)


def _sc_deg_body(src_hbm, dst_hbm, degp_hbm, sbuf, dbuf, hist, claim):
    wid = _wid()
    base = wid * SHARD
    pltpu.sync_copy(src_hbm.at[pl.ds(base, SHARD)], sbuf)
    pltpu.sync_copy(dst_hbm.at[pl.ds(base, SHARD)], dbuf)
    _zero_ref(hist, 2 * NP)
    lane = lax.iota(jnp.int32, LANES)
    ones = jnp.ones((LANES,), jnp.float32)

    def body(v, _):
        s = sbuf[pl.ds(v * LANES, LANES)]
        plsc.store_scatter(claim, (s,), lane)
        won = plsc.load_gather(claim, (s,))
        safe = won == lane
        plsc.addupdate_scatter(hist, (s,), ones, mask=safe)
        rem = jnp.logical_not(safe)

        def cond(r):
            return jnp.any(r)

        def tail(r):
            plsc.store_scatter(claim, (s,), lane, mask=r)
            w2 = plsc.load_gather(claim, (s,), mask=r)
            ok = jnp.logical_and(r, w2 == lane)
            plsc.addupdate_scatter(hist, (s,), ones, mask=ok)
            return jnp.logical_and(r, jnp.logical_not(ok))

        lax.while_loop(cond, tail, rem)

        d = dbuf[pl.ds(v * LANES, LANES)]
        dt = d + NP
        plsc.store_scatter(claim, (d,), lane)
        wond = plsc.load_gather(claim, (d,))
        safed = wond == lane
        plsc.addupdate_scatter(hist, (dt,), ones, mask=safed)
        remd = jnp.logical_not(safed)

        def taild(r):
            plsc.store_scatter(claim, (d,), lane, mask=r)
            w2 = plsc.load_gather(claim, (d,), mask=r)
            ok = jnp.logical_and(r, w2 == lane)
            plsc.addupdate_scatter(hist, (dt,), ones, mask=ok)
            return jnp.logical_and(r, jnp.logical_not(ok))

        lax.while_loop(cond, taild, remd)
        return 0

    lax.fori_loop(0, SHARD // LANES, body, 0)
    pltpu.sync_copy(hist.at[pl.ds(0, NP)], degp_hbm.at[pl.ds(wid * NP, NP)])
    pltpu.sync_copy(hist.at[pl.ds(NP, NP)],
                    degp_hbm.at[pl.ds(NW * NP + wid * NP, NP)])


def _sc_deg(src, dst):
    mesh = plsc.VectorSubcoreMesh(core_axis_name="c", subcore_axis_name="s")
    k = pl.kernel(
        _sc_deg_body,
        out_type=jax.ShapeDtypeStruct((2 * NW * NP,), jnp.float32),
        mesh=mesh,
        compiler_params=pltpu.CompilerParams(needs_layout_passes=False),
        scratch_types=[
            pltpu.VMEM((SHARD,), jnp.int32),
            pltpu.VMEM((SHARD,), jnp.int32),
            pltpu.VMEM((2 * NP,), jnp.float32),
            pltpu.VMEM((NP,), jnp.int32),
        ],
    )
    return k(src, dst)


def _sc_layer_body(y_hbm, src_hbm, dst_hbm, agg_hbm, yslab, aslab, sbuf, dbuf,
                   claim):
    wid = _wid()
    fbase = wid * FPT * NP
    pltpu.sync_copy(y_hbm.at[pl.ds(fbase, FPT * NP)], yslab)
    _zero_ref(aslab, FPT * NP)
    lane = lax.iota(jnp.int32, LANES)

    def chunk(g, _):
        ebase = g * CH
        pltpu.sync_copy(src_hbm.at[pl.ds(ebase, CH)], sbuf)
        pltpu.sync_copy(dst_hbm.at[pl.ds(ebase, CH)], dbuf)

        def vbody(v, _):
            s = sbuf[pl.ds(v * LANES, LANES)]
            d = dbuf[pl.ds(v * LANES, LANES)]
            plsc.store_scatter(claim, (d,), lane)
            won = plsc.load_gather(claim, (d,))
            safe = won == lane
            vals = []
            for f in range(FPT):
                vals.append(plsc.load_gather(yslab, (s + f * NP,)))
            for f in range(FPT):
                plsc.addupdate_scatter(aslab, (d + f * NP,), vals[f],
                                       mask=safe)
            rem = jnp.logical_not(safe)

            def cond(r):
                return jnp.any(r)

            def tail(r):
                plsc.store_scatter(claim, (d,), lane, mask=r)
                w2 = plsc.load_gather(claim, (d,), mask=r)
                ok = jnp.logical_and(r, w2 == lane)
                for f in range(FPT):
                    gv = plsc.load_gather(yslab, (s + f * NP,), mask=ok)
                    plsc.addupdate_scatter(aslab, (d + f * NP,), gv, mask=ok)
                return jnp.logical_and(r, jnp.logical_not(ok))

            lax.while_loop(cond, tail, rem)
            return 0

        lax.fori_loop(0, VPC, vbody, 0)
        return 0

    lax.fori_loop(0, NCH, chunk, 0)
    pltpu.sync_copy(aslab, agg_hbm.at[pl.ds(fbase, FPT * NP)])


def _sc_layer(y_flat, src, dst):
    mesh = plsc.VectorSubcoreMesh(core_axis_name="c", subcore_axis_name="s")
    k = pl.kernel(
        _sc_layer_body,
        out_type=jax.ShapeDtypeStruct((D * NP,), jnp.float32),
        mesh=mesh,
        compiler_params=pltpu.CompilerParams(needs_layout_passes=False),
        scratch_types=[
            pltpu.VMEM((FPT * NP,), jnp.float32),
            pltpu.VMEM((FPT * NP,), jnp.float32),
            pltpu.VMEM((CH,), jnp.int32),
            pltpu.VMEM((CH,), jnp.int32),
            pltpu.VMEM((NP,), jnp.int32),
        ],
    )
    return k(y_flat, src, dst)


def _tc1_body(h_ref, w1_ref, degp_ref, y_ref, on_ref, in_ref):
    deg = jnp.sum(degp_ref[...], axis=1)
    onorm = lax.rsqrt(jnp.maximum(deg[0:1, :], 1.0))
    inorm = lax.rsqrt(jnp.maximum(deg[1:2, :], 1.0))
    y = lax.dot_general(w1_ref[...], h_ref[...], (((0,), (1,)), ((), ())),
                        preferred_element_type=jnp.float32)
    y_ref[...] = y * onorm
    on_ref[...] = onorm
    in_ref[...] = inorm


def _tc1(h_pad, w1, degp):
    return pl.pallas_call(
        _tc1_body,
        grid=(GRID,),
        in_specs=[
            pl.BlockSpec((BN, D), lambda i: (i, 0)),
            pl.BlockSpec((D, D), lambda i: (0, 0)),
            pl.BlockSpec((2, NW, BN), lambda i: (0, 0, i)),
        ],
        out_specs=[
            pl.BlockSpec((D, BN), lambda i: (0, i)),
            pl.BlockSpec((1, BN), lambda i: (0, i)),
            pl.BlockSpec((1, BN), lambda i: (0, i)),
        ],
        out_shape=[
            jax.ShapeDtypeStruct((D, NP), jnp.float32),
            jax.ShapeDtypeStruct((1, NP), jnp.float32),
            jax.ShapeDtypeStruct((1, NP), jnp.float32),
        ],
    )(h_pad, w1, degp)


def _tc_mid_body(agg_ref, in_ref, on_ref, b_ref, w_ref, y_ref):
    hprev = jnp.maximum(agg_ref[...] * in_ref[...] + b_ref[...], 0.0)
    y = lax.dot_general(w_ref[...], hprev, (((0,), (0,)), ((), ())),
                        preferred_element_type=jnp.float32)
    y_ref[...] = y * on_ref[...]


def _tc_mid(agg, inorm, onorm, b_prev, w):
    return pl.pallas_call(
        _tc_mid_body,
        grid=(GRID,),
        in_specs=[
            pl.BlockSpec((D, BN), lambda i: (0, i)),
            pl.BlockSpec((1, BN), lambda i: (0, i)),
            pl.BlockSpec((1, BN), lambda i: (0, i)),
            pl.BlockSpec((D, 1), lambda i: (0, 0)),
            pl.BlockSpec((D, D), lambda i: (0, 0)),
        ],
        out_specs=pl.BlockSpec((D, BN), lambda i: (0, i)),
        out_shape=jax.ShapeDtypeStruct((D, NP), jnp.float32),
    )(agg, inorm, onorm, b_prev, w)


def _tc_fin_body(agg_ref, in_ref, b_ref, o_ref):
    i = pl.program_id(0)
    h3 = jnp.maximum(agg_ref[...] * in_ref[...] + b_ref[...], 0.0)
    col = lax.broadcasted_iota(jnp.int32, (1, BN), 1) + i * BN
    h3 = jnp.where(col < N, h3, 0.0)
    part = jnp.sum(h3, axis=1)

    @pl.when(i == 0)
    def _():
        o_ref[...] = jnp.zeros_like(o_ref)

    o_ref[...] += part[None, :]

    @pl.when(i == GRID - 1)
    def _():
        o_ref[...] *= jnp.float32(1.0 / N)


def _tc_fin(agg3, inorm, b3):
    return pl.pallas_call(
        _tc_fin_body,
        grid=(GRID,),
        in_specs=[
            pl.BlockSpec((D, BN), lambda i: (0, i)),
            pl.BlockSpec((1, BN), lambda i: (0, i)),
            pl.BlockSpec((D, 1), lambda i: (0, 0)),
        ],
        out_specs=pl.BlockSpec((1, D), lambda i: (0, 0)),
        out_shape=jax.ShapeDtypeStruct((1, D), jnp.float32),
    )(agg3, inorm, b3)


def kernel(h, edge_index, W1, b1, W2, b2, W3, b3):
    src = edge_index[0]
    dst = edge_index[1]
    h_pad = jnp.pad(h, ((0, NP - N), (0, 0)))

    degp = _sc_deg(src, dst).reshape(2, NW, NP)
    y1, onorm, inorm = _tc1(h_pad, W1, degp)

    agg1 = _sc_layer(y1.reshape(-1), src, dst).reshape(D, NP)
    y2 = _tc_mid(agg1, inorm, onorm, b1.reshape(D, 1), W2)

    agg2 = _sc_layer(y2.reshape(-1), src, dst).reshape(D, NP)
    y3 = _tc_mid(agg2, inorm, onorm, b2.reshape(D, 1), W3)

    agg3 = _sc_layer(y3.reshape(-1), src, dst).reshape(D, NP)
    return _tc_fin(agg3, inorm, b3.reshape(D, 1))


# speedup vs baseline: 2.1907x; 1.0048x over previous
import functools

import jax
import jax.numpy as jnp
from jax import lax
from jax.experimental import pallas as pl
from jax.experimental.pallas import tpu as pltpu
from jax.experimental.pallas import tpu_sc as plsc

N = 10000
NP = 10240
E = 320000
D = 128
NW = 32
FPT = D // NW
CH = 4000
NCH = E // CH
SHARD = E // NW
BN = 1280
GRID = NP // BN
VPC = CH // 16
LANES = 16
UNROLL = 5


def _wid():
    return lax.axis_index("s") * 2 + lax.axis_index("c")


def _scatter_add_resolved(acc_ref, idx, val, claim_ref, lane):
    plsc.store_scatter(claim_ref, (idx,), lane)
    won = plsc.load_gather(claim_ref, (idx,))
    safe = won == lane
    plsc.addupdate_scatter(acc_ref, (idx,), val, mask=safe)
    rem = jnp.logical_not(safe)

    def cond(r):
        return jnp.any(r)

    def tail(r):
        plsc.store_scatter(claim_ref, (idx,), lane, mask=r)
        w2 = plsc.load_gather(claim_ref, (idx,), mask=r)
        ok = jnp.logical_and(r, w2 == lane)
        plsc.addupdate_scatter(acc_ref, (idx,), val, mask=ok)
        return jnp.logical_and(r, jnp.logical_not(ok))

    lax.while_loop(cond, tail, rem)


def _zero_ref(ref, nwords):
    z = jnp.zeros((LANES,), jnp.float32)

    def body(i, _):
        ref[pl.ds(i * LANES, LANES)] = z
        return 0

    lax.fori_loop(0, nwords // LANES, body, 0)


def _sc_deg_body(src_hbm, dst_hbm, degp_hbm, sbuf, dbuf, hist, claim):
    wid = _wid()
    base = wid * SHARD
    pltpu.sync_copy(src_hbm.at[pl.ds(base, SHARD)], sbuf)
    pltpu.sync_copy(dst_hbm.at[pl.ds(base, SHARD)], dbuf)
    _zero_ref(hist, 2 * NP)
    lane = lax.iota(jnp.int32, LANES)
    ones = jnp.ones((LANES,), jnp.float32)

    def body(v, _):
        s = sbuf[pl.ds(v * LANES, LANES)]
        plsc.store_scatter(claim, (s,), lane)
        won = plsc.load_gather(claim, (s,))
        safe = won == lane
        plsc.addupdate_scatter(hist, (s,), ones, mask=safe)
        rem = jnp.logical_not(safe)

        def cond(r):
            return jnp.any(r)

        def tail(r):
            plsc.store_scatter(claim, (s,), lane, mask=r)
            w2 = plsc.load_gather(claim, (s,), mask=r)
            ok = jnp.logical_and(r, w2 == lane)
            plsc.addupdate_scatter(hist, (s,), ones, mask=ok)
            return jnp.logical_and(r, jnp.logical_not(ok))

        lax.while_loop(cond, tail, rem)

        d = dbuf[pl.ds(v * LANES, LANES)]
        dt = d + NP
        plsc.store_scatter(claim, (d,), lane)
        wond = plsc.load_gather(claim, (d,))
        safed = wond == lane
        plsc.addupdate_scatter(hist, (dt,), ones, mask=safed)
        remd = jnp.logical_not(safed)

        def taild(r):
            plsc.store_scatter(claim, (d,), lane, mask=r)
            w2 = plsc.load_gather(claim, (d,), mask=r)
            ok = jnp.logical_and(r, w2 == lane)
            plsc.addupdate_scatter(hist, (dt,), ones, mask=ok)
            return jnp.logical_and(r, jnp.logical_not(ok))

        lax.while_loop(cond, taild, remd)
        return 0

    lax.fori_loop(0, SHARD // LANES, body, 0)
    pltpu.sync_copy(hist.at[pl.ds(0, NP)], degp_hbm.at[pl.ds(wid * NP, NP)])
    pltpu.sync_copy(hist.at[pl.ds(NP, NP)],
                    degp_hbm.at[pl.ds(NW * NP + wid * NP, NP)])


def _sc_deg(src, dst):
    mesh = plsc.VectorSubcoreMesh(core_axis_name="c", subcore_axis_name="s")
    k = pl.kernel(
        _sc_deg_body,
        out_type=jax.ShapeDtypeStruct((2 * NW * NP,), jnp.float32),
        mesh=mesh,
        compiler_params=pltpu.CompilerParams(needs_layout_passes=False),
        scratch_types=[
            pltpu.VMEM((SHARD,), jnp.int32),
            pltpu.VMEM((SHARD,), jnp.int32),
            pltpu.VMEM((2 * NP,), jnp.float32),
            pltpu.VMEM((NP,), jnp.int32),
        ],
    )
    return k(src, dst)


def _sc_layer_body(y_hbm, src_hbm, dst_hbm, agg_hbm, yslab, aslab, sbuf, dbuf,
                   claim):
    wid = _wid()
    fbase = wid * FPT * NP
    pltpu.sync_copy(y_hbm.at[pl.ds(fbase, FPT * NP)], yslab)
    _zero_ref(aslab, FPT * NP)
    lane = lax.iota(jnp.int32, LANES)

    def chunk(g, _):
        ebase = g * CH
        pltpu.sync_copy(src_hbm.at[pl.ds(ebase, CH)], sbuf)
        pltpu.sync_copy(dst_hbm.at[pl.ds(ebase, CH)], dbuf)

        def vblock(blk, _):
            svecs, dvecs, rems = [], [], []
            for k in range(UNROLL):
                v = blk * UNROLL + k
                s = sbuf[pl.ds(v * LANES, LANES)]
                d = dbuf[pl.ds(v * LANES, LANES)]
                plsc.store_scatter(claim, (d,), lane)
                won = plsc.load_gather(claim, (d,))
                safe = won == lane
                for f in range(FPT):
                    gv = plsc.load_gather(yslab, (s + f * NP,))
                    plsc.addupdate_scatter(aslab, (d + f * NP,), gv,
                                           mask=safe)
                svecs.append(s)
                dvecs.append(d)
                rems.append(jnp.logical_not(safe))

            anyrem = rems[0]
            for k in range(1, UNROLL):
                anyrem = jnp.logical_or(anyrem, rems[k])

            def cond(rs):
                acc = rs[0]
                for k in range(1, UNROLL):
                    acc = jnp.logical_or(acc, rs[k])
                return jnp.any(acc)

            def tail(rs):
                out = []
                for k in range(UNROLL):
                    r = rs[k]
                    plsc.store_scatter(claim, (dvecs[k],), lane, mask=r)
                    w2 = plsc.load_gather(claim, (dvecs[k],), mask=r)
                    ok = jnp.logical_and(r, w2 == lane)
                    for f in range(FPT):
                        gv = plsc.load_gather(yslab, (svecs[k] + f * NP,),
                                              mask=ok)
                        plsc.addupdate_scatter(aslab, (dvecs[k] + f * NP,),
                                               gv, mask=ok)
                    out.append(jnp.logical_and(r, jnp.logical_not(ok)))
                return tuple(out)

            lax.while_loop(cond, tail, tuple(rems))
            return 0

        lax.fori_loop(0, VPC // UNROLL, vblock, 0)
        return 0

    lax.fori_loop(0, NCH, chunk, 0)
    pltpu.sync_copy(aslab, agg_hbm.at[pl.ds(fbase, FPT * NP)])


def _sc_layer(y_flat, src, dst):
    mesh = plsc.VectorSubcoreMesh(core_axis_name="c", subcore_axis_name="s")
    k = pl.kernel(
        _sc_layer_body,
        out_type=jax.ShapeDtypeStruct((D * NP,), jnp.float32),
        mesh=mesh,
        compiler_params=pltpu.CompilerParams(needs_layout_passes=False),
        scratch_types=[
            pltpu.VMEM((FPT * NP,), jnp.float32),
            pltpu.VMEM((FPT * NP,), jnp.float32),
            pltpu.VMEM((CH,), jnp.int32),
            pltpu.VMEM((CH,), jnp.int32),
            pltpu.VMEM((NP,), jnp.int32),
        ],
    )
    return k(y_flat, src, dst)


def _tc1_body(h_ref, w1_ref, degp_ref, y_ref, on_ref, in_ref):
    deg = jnp.sum(degp_ref[...], axis=1)
    onorm = lax.rsqrt(jnp.maximum(deg[0:1, :], 1.0))
    inorm = lax.rsqrt(jnp.maximum(deg[1:2, :], 1.0))
    y = lax.dot_general(w1_ref[...], h_ref[...], (((0,), (1,)), ((), ())),
                        preferred_element_type=jnp.float32)
    y_ref[...] = y * onorm
    on_ref[...] = onorm
    in_ref[...] = inorm


def _tc1(h_pad, w1, degp):
    return pl.pallas_call(
        _tc1_body,
        grid=(GRID,),
        in_specs=[
            pl.BlockSpec((BN, D), lambda i: (i, 0)),
            pl.BlockSpec((D, D), lambda i: (0, 0)),
            pl.BlockSpec((2, NW, BN), lambda i: (0, 0, i)),
        ],
        out_specs=[
            pl.BlockSpec((D, BN), lambda i: (0, i)),
            pl.BlockSpec((1, BN), lambda i: (0, i)),
            pl.BlockSpec((1, BN), lambda i: (0, i)),
        ],
        out_shape=[
            jax.ShapeDtypeStruct((D, NP), jnp.float32),
            jax.ShapeDtypeStruct((1, NP), jnp.float32),
            jax.ShapeDtypeStruct((1, NP), jnp.float32),
        ],
    )(h_pad, w1, degp)


def _tc_mid_body(agg_ref, in_ref, on_ref, b_ref, w_ref, y_ref):
    hprev = jnp.maximum(agg_ref[...] * in_ref[...] + b_ref[...], 0.0)
    y = lax.dot_general(w_ref[...], hprev, (((0,), (0,)), ((), ())),
                        preferred_element_type=jnp.float32)
    y_ref[...] = y * on_ref[...]


def _tc_mid(agg, inorm, onorm, b_prev, w):
    return pl.pallas_call(
        _tc_mid_body,
        grid=(GRID,),
        in_specs=[
            pl.BlockSpec((D, BN), lambda i: (0, i)),
            pl.BlockSpec((1, BN), lambda i: (0, i)),
            pl.BlockSpec((1, BN), lambda i: (0, i)),
            pl.BlockSpec((D, 1), lambda i: (0, 0)),
            pl.BlockSpec((D, D), lambda i: (0, 0)),
        ],
        out_specs=pl.BlockSpec((D, BN), lambda i: (0, i)),
        out_shape=jax.ShapeDtypeStruct((D, NP), jnp.float32),
    )(agg, inorm, onorm, b_prev, w)


def _tc_fin_body(agg_ref, in_ref, b_ref, o_ref):
    i = pl.program_id(0)
    h3 = jnp.maximum(agg_ref[...] * in_ref[...] + b_ref[...], 0.0)
    col = lax.broadcasted_iota(jnp.int32, (1, BN), 1) + i * BN
    h3 = jnp.where(col < N, h3, 0.0)
    part = jnp.sum(h3, axis=1)

    @pl.when(i == 0)
    def _():
        o_ref[...] = jnp.zeros_like(o_ref)

    o_ref[...] += part[None, :]

    @pl.when(i == GRID - 1)
    def _():
        o_ref[...] *= jnp.float32(1.0 / N)


def _tc_fin(agg3, inorm, b3):
    return pl.pallas_call(
        _tc_fin_body,
        grid=(GRID,),
        in_specs=[
            pl.BlockSpec((D, BN), lambda i: (0, i)),
            pl.BlockSpec((1, BN), lambda i: (0, i)),
            pl.BlockSpec((D, 1), lambda i: (0, 0)),
        ],
        out_specs=pl.BlockSpec((1, D), lambda i: (0, 0)),
        out_shape=jax.ShapeDtypeStruct((1, D), jnp.float32),
    )(agg3, inorm, b3)


def kernel(h, edge_index, W1, b1, W2, b2, W3, b3):
    src = edge_index[0]
    dst = edge_index[1]
    h_pad = jnp.pad(h, ((0, NP - N), (0, 0)))

    degp = _sc_deg(src, dst).reshape(2, NW, NP)
    y1, onorm, inorm = _tc1(h_pad, W1, degp)

    agg1 = _sc_layer(y1.reshape(-1), src, dst).reshape(D, NP)
    y2 = _tc_mid(agg1, inorm, onorm, b1.reshape(D, 1), W2)

    agg2 = _sc_layer(y2.reshape(-1), src, dst).reshape(D, NP)
    y3 = _tc_mid(agg2, inorm, onorm, b2.reshape(D, 1), W3)

    agg3 = _sc_layer(y3.reshape(-1), src, dst).reshape(D, NP)
    return _tc_fin(agg3, inorm, b3.reshape(D, 1))


# speedup vs baseline: 2.7340x; 1.2480x over previous
import functools

import jax
import jax.numpy as jnp
from jax import lax
from jax.experimental import pallas as pl
from jax.experimental.pallas import tpu as pltpu
from jax.experimental.pallas import tpu_sc as plsc

N = 10000
NP = 10240
E = 320000
D = 128
NW = 32
FPT = D // NW
CH = 4000
NCH = E // CH
SHARD = E // NW
BN = 1280
GRID = NP // BN
VPC = CH // 16
LANES = 16
UNROLL = 8
G = 10


def _wid():
    return lax.axis_index("s") * 2 + lax.axis_index("c")


def _zero_ref(ref, nwords):
    z = jnp.zeros((LANES,), jnp.float32)

    def body(i, _):
        ref[pl.ds(i * LANES, LANES)] = z
        return 0

    lax.fori_loop(0, nwords // LANES, body, 0)


def _sc_deg_body(src_hbm, dst_hbm, degp_hbm, sbuf, dbuf, hist, claim):
    wid = _wid()
    base = wid * SHARD
    pltpu.sync_copy(src_hbm.at[pl.ds(base, SHARD)], sbuf)
    pltpu.sync_copy(dst_hbm.at[pl.ds(base, SHARD)], dbuf)
    _zero_ref(hist, 2 * NP)
    lane = lax.iota(jnp.int32, LANES)
    ones = jnp.ones((LANES,), jnp.float32)

    def body(v, _):
        s = sbuf[pl.ds(v * LANES, LANES)]
        plsc.store_scatter(claim, (s,), lane)
        won = plsc.load_gather(claim, (s,))
        safe = won == lane
        plsc.addupdate_scatter(hist, (s,), ones, mask=safe)
        rem = jnp.logical_not(safe)

        def cond(r):
            return jnp.any(r)

        def tail(r):
            plsc.store_scatter(claim, (s,), lane, mask=r)
            w2 = plsc.load_gather(claim, (s,), mask=r)
            ok = jnp.logical_and(r, w2 == lane)
            plsc.addupdate_scatter(hist, (s,), ones, mask=ok)
            return jnp.logical_and(r, jnp.logical_not(ok))

        lax.while_loop(cond, tail, rem)

        d = dbuf[pl.ds(v * LANES, LANES)]
        dt = d + NP
        plsc.store_scatter(claim, (d,), lane)
        wond = plsc.load_gather(claim, (d,))
        safed = wond == lane
        plsc.addupdate_scatter(hist, (dt,), ones, mask=safed)
        remd = jnp.logical_not(safed)

        def taild(r):
            plsc.store_scatter(claim, (d,), lane, mask=r)
            w2 = plsc.load_gather(claim, (d,), mask=r)
            ok = jnp.logical_and(r, w2 == lane)
            plsc.addupdate_scatter(hist, (dt,), ones, mask=ok)
            return jnp.logical_and(r, jnp.logical_not(ok))

        lax.while_loop(cond, taild, remd)
        return 0

    lax.fori_loop(0, SHARD // LANES, body, 0)
    pltpu.sync_copy(hist.at[pl.ds(0, NP)], degp_hbm.at[pl.ds(wid * NP, NP)])
    pltpu.sync_copy(hist.at[pl.ds(NP, NP)],
                    degp_hbm.at[pl.ds(NW * NP + wid * NP, NP)])


def _sc_deg(src, dst):
    mesh = plsc.VectorSubcoreMesh(core_axis_name="c", subcore_axis_name="s")
    k = pl.kernel(
        _sc_deg_body,
        out_type=jax.ShapeDtypeStruct((2 * NW * NP,), jnp.float32),
        mesh=mesh,
        compiler_params=pltpu.CompilerParams(needs_layout_passes=False),
        scratch_types=[
            pltpu.VMEM((SHARD,), jnp.int32),
            pltpu.VMEM((SHARD,), jnp.int32),
            pltpu.VMEM((2 * NP,), jnp.float32),
            pltpu.VMEM((NP,), jnp.int32),
        ],
    )
    return k(src, dst)


def _sc_layer_body(y_hbm, src_hbm, dst_hbm, agg_hbm, yslab, aslab, sbuf, dbuf,
                   claim, mbuf):
    wid = _wid()
    fbase = wid * FPT * NP
    pltpu.sync_copy(y_hbm.at[pl.ds(fbase, FPT * NP)], yslab)
    _zero_ref(aslab, FPT * NP)
    lane = lax.iota(jnp.int32, LANES)
    zero16 = jnp.zeros((LANES,), jnp.int32)
    one16 = jnp.ones((LANES,), jnp.int32)

    def chunk(g, _):
        ebase = g * CH
        pltpu.sync_copy(src_hbm.at[pl.ds(ebase, CH)], sbuf)
        pltpu.sync_copy(dst_hbm.at[pl.ds(ebase, CH)], dbuf)

        @plsc.parallel_loop(0, VPC, unroll=UNROLL)
        def hot(v):
            s = sbuf[pl.ds(v * LANES, LANES)]
            d = dbuf[pl.ds(v * LANES, LANES)]
            plsc.store_scatter(claim, (d,), lane)
            won = plsc.load_gather(claim, (d,))
            safe = won == lane
            for f in range(FPT):
                gv = plsc.load_gather(yslab, (s + f * NP,))
                plsc.addupdate_scatter(aslab, (d + f * NP,), gv, mask=safe)
            mbuf[pl.ds(v * LANES, LANES)] = jnp.where(safe, zero16, one16)

        def grp(gi, _):
            acc = mbuf[pl.ds(gi * G * LANES, LANES)]
            for k in range(1, G):
                acc = acc | mbuf[pl.ds((gi * G + k) * LANES, LANES)]

            @pl.when(jnp.any(acc != 0))
            def _():
                def per_vreg(vv, _):
                    v = gi * G + vv
                    lose = mbuf[pl.ds(v * LANES, LANES)] != 0

                    @pl.when(jnp.any(lose))
                    def _():
                        s = sbuf[pl.ds(v * LANES, LANES)]
                        d = dbuf[pl.ds(v * LANES, LANES)]

                        def cond(r):
                            return jnp.any(r)

                        def tail(r):
                            plsc.store_scatter(claim, (d,), lane, mask=r)
                            w2 = plsc.load_gather(claim, (d,), mask=r)
                            ok = jnp.logical_and(r, w2 == lane)
                            for f in range(FPT):
                                gv = plsc.load_gather(yslab, (s + f * NP,),
                                                      mask=ok)
                                plsc.addupdate_scatter(aslab, (d + f * NP,),
                                                       gv, mask=ok)
                            return jnp.logical_and(r, jnp.logical_not(ok))

                        lax.while_loop(cond, tail, lose)

                    return 0

                lax.fori_loop(0, G, per_vreg, 0)

            return 0

        lax.fori_loop(0, VPC // G, grp, 0)
        return 0

    lax.fori_loop(0, NCH, chunk, 0)
    pltpu.sync_copy(aslab, agg_hbm.at[pl.ds(fbase, FPT * NP)])


def _sc_layer(y_flat, src, dst):
    mesh = plsc.VectorSubcoreMesh(core_axis_name="c", subcore_axis_name="s")
    k = pl.kernel(
        _sc_layer_body,
        out_type=jax.ShapeDtypeStruct((D * NP,), jnp.float32),
        mesh=mesh,
        compiler_params=pltpu.CompilerParams(needs_layout_passes=False),
        scratch_types=[
            pltpu.VMEM((FPT * NP,), jnp.float32),
            pltpu.VMEM((FPT * NP,), jnp.float32),
            pltpu.VMEM((CH,), jnp.int32),
            pltpu.VMEM((CH,), jnp.int32),
            pltpu.VMEM((NP,), jnp.int32),
            pltpu.VMEM((CH,), jnp.int32),
        ],
    )
    return k(y_flat, src, dst)


def _tc1_body(h_ref, w1_ref, degp_ref, y_ref, on_ref, in_ref):
    deg = jnp.sum(degp_ref[...], axis=1)
    onorm = lax.rsqrt(jnp.maximum(deg[0:1, :], 1.0))
    inorm = lax.rsqrt(jnp.maximum(deg[1:2, :], 1.0))
    y = lax.dot_general(w1_ref[...], h_ref[...], (((0,), (1,)), ((), ())),
                        preferred_element_type=jnp.float32)
    y_ref[...] = y * onorm
    on_ref[...] = onorm
    in_ref[...] = inorm


def _tc1(h_pad, w1, degp):
    return pl.pallas_call(
        _tc1_body,
        grid=(GRID,),
        in_specs=[
            pl.BlockSpec((BN, D), lambda i: (i, 0)),
            pl.BlockSpec((D, D), lambda i: (0, 0)),
            pl.BlockSpec((2, NW, BN), lambda i: (0, 0, i)),
        ],
        out_specs=[
            pl.BlockSpec((D, BN), lambda i: (0, i)),
            pl.BlockSpec((1, BN), lambda i: (0, i)),
            pl.BlockSpec((1, BN), lambda i: (0, i)),
        ],
        out_shape=[
            jax.ShapeDtypeStruct((D, NP), jnp.float32),
            jax.ShapeDtypeStruct((1, NP), jnp.float32),
            jax.ShapeDtypeStruct((1, NP), jnp.float32),
        ],
    )(h_pad, w1, degp)


def _tc_mid_body(agg_ref, in_ref, on_ref, b_ref, w_ref, y_ref):
    hprev = jnp.maximum(agg_ref[...] * in_ref[...] + b_ref[...], 0.0)
    y = lax.dot_general(w_ref[...], hprev, (((0,), (0,)), ((), ())),
                        preferred_element_type=jnp.float32)
    y_ref[...] = y * on_ref[...]


def _tc_mid(agg, inorm, onorm, b_prev, w):
    return pl.pallas_call(
        _tc_mid_body,
        grid=(GRID,),
        in_specs=[
            pl.BlockSpec((D, BN), lambda i: (0, i)),
            pl.BlockSpec((1, BN), lambda i: (0, i)),
            pl.BlockSpec((1, BN), lambda i: (0, i)),
            pl.BlockSpec((D, 1), lambda i: (0, 0)),
            pl.BlockSpec((D, D), lambda i: (0, 0)),
        ],
        out_specs=pl.BlockSpec((D, BN), lambda i: (0, i)),
        out_shape=jax.ShapeDtypeStruct((D, NP), jnp.float32),
    )(agg, inorm, onorm, b_prev, w)


def _tc_fin_body(agg_ref, in_ref, b_ref, o_ref):
    i = pl.program_id(0)
    h3 = jnp.maximum(agg_ref[...] * in_ref[...] + b_ref[...], 0.0)
    col = lax.broadcasted_iota(jnp.int32, (1, BN), 1) + i * BN
    h3 = jnp.where(col < N, h3, 0.0)
    part = jnp.sum(h3, axis=1)

    @pl.when(i == 0)
    def _():
        o_ref[...] = jnp.zeros_like(o_ref)

    o_ref[...] += part[None, :]

    @pl.when(i == GRID - 1)
    def _():
        o_ref[...] *= jnp.float32(1.0 / N)


def _tc_fin(agg3, inorm, b3):
    return pl.pallas_call(
        _tc_fin_body,
        grid=(GRID,),
        in_specs=[
            pl.BlockSpec((D, BN), lambda i: (0, i)),
            pl.BlockSpec((1, BN), lambda i: (0, i)),
            pl.BlockSpec((D, 1), lambda i: (0, 0)),
        ],
        out_specs=pl.BlockSpec((1, D), lambda i: (0, 0)),
        out_shape=jax.ShapeDtypeStruct((1, D), jnp.float32),
    )(agg3, inorm, b3)


def kernel(h, edge_index, W1, b1, W2, b2, W3, b3):
    src = edge_index[0]
    dst = edge_index[1]
    h_pad = jnp.pad(h, ((0, NP - N), (0, 0)))

    degp = _sc_deg(src, dst).reshape(2, NW, NP)
    y1, onorm, inorm = _tc1(h_pad, W1, degp)

    agg1 = _sc_layer(y1.reshape(-1), src, dst).reshape(D, NP)
    y2 = _tc_mid(agg1, inorm, onorm, b1.reshape(D, 1), W2)

    agg2 = _sc_layer(y2.reshape(-1), src, dst).reshape(D, NP)
    y3 = _tc_mid(agg2, inorm, onorm, b2.reshape(D, 1), W3)

    agg3 = _sc_layer(y3.reshape(-1), src, dst).reshape(D, NP)
    return _tc_fin(agg3, inorm, b3.reshape(D, 1))


# speedup vs baseline: 3.3053x; 1.2090x over previous
import functools

import jax
import jax.numpy as jnp
from jax import lax
from jax.experimental import pallas as pl
from jax.experimental.pallas import tpu as pltpu
from jax.experimental.pallas import tpu_sc as plsc

N = 10000
NP = 10240
E = 320000
D = 128
NW = 32
FPT = D // NW
CH = 4000
NCH = E // CH
SHARD = E // NW
BN = 1280
GRID = NP // BN
VPC = CH // 16
LANES = 16
UNROLL = 8
G = 10


def _wid():
    return lax.axis_index("s") * 2 + lax.axis_index("c")


def _zero_ref(ref, nwords):
    z = jnp.zeros((LANES,), jnp.float32)

    def body(i, _):
        ref[pl.ds(i * LANES, LANES)] = z
        return 0

    lax.fori_loop(0, nwords // LANES, body, 0)


def _sc_deg_body(src_hbm, dst_hbm, degp_hbm, packed_hbm, sbuf, dbuf, hist,
                 claim, pbuf):
    wid = _wid()
    base = wid * SHARD
    pltpu.sync_copy(src_hbm.at[pl.ds(base, SHARD)], sbuf)
    pltpu.sync_copy(dst_hbm.at[pl.ds(base, SHARD)], dbuf)
    _zero_ref(hist, 2 * NP)
    lane = lax.iota(jnp.int32, LANES)
    ones = jnp.ones((LANES,), jnp.float32)

    def body(v, _):
        s = sbuf[pl.ds(v * LANES, LANES)]
        plsc.store_scatter(claim, (s,), lane)
        won = plsc.load_gather(claim, (s,))
        safe = won == lane
        plsc.addupdate_scatter(hist, (s,), ones, mask=safe)
        rem = jnp.logical_not(safe)

        def cond(r):
            return jnp.any(r)

        def tail(r):
            plsc.store_scatter(claim, (s,), lane, mask=r)
            w2 = plsc.load_gather(claim, (s,), mask=r)
            ok = jnp.logical_and(r, w2 == lane)
            plsc.addupdate_scatter(hist, (s,), ones, mask=ok)
            return jnp.logical_and(r, jnp.logical_not(ok))

        lax.while_loop(cond, tail, rem)

        d = dbuf[pl.ds(v * LANES, LANES)]
        pbuf[pl.ds(v * LANES, LANES)] = s | (d << 14)
        dt = d + NP
        plsc.store_scatter(claim, (d,), lane)
        wond = plsc.load_gather(claim, (d,))
        safed = wond == lane
        plsc.addupdate_scatter(hist, (dt,), ones, mask=safed)
        remd = jnp.logical_not(safed)

        def taild(r):
            plsc.store_scatter(claim, (d,), lane, mask=r)
            w2 = plsc.load_gather(claim, (d,), mask=r)
            ok = jnp.logical_and(r, w2 == lane)
            plsc.addupdate_scatter(hist, (dt,), ones, mask=ok)
            return jnp.logical_and(r, jnp.logical_not(ok))

        lax.while_loop(cond, taild, remd)
        return 0

    lax.fori_loop(0, SHARD // LANES, body, 0)
    pltpu.sync_copy(hist.at[pl.ds(0, NP)], degp_hbm.at[pl.ds(wid * NP, NP)])
    pltpu.sync_copy(hist.at[pl.ds(NP, NP)],
                    degp_hbm.at[pl.ds(NW * NP + wid * NP, NP)])
    pltpu.sync_copy(pbuf, packed_hbm.at[pl.ds(base, SHARD)])


def _sc_deg(src, dst):
    mesh = plsc.VectorSubcoreMesh(core_axis_name="c", subcore_axis_name="s")
    k = pl.kernel(
        _sc_deg_body,
        out_type=[
            jax.ShapeDtypeStruct((2 * NW * NP,), jnp.float32),
            jax.ShapeDtypeStruct((E,), jnp.int32),
        ],
        mesh=mesh,
        compiler_params=pltpu.CompilerParams(needs_layout_passes=False),
        scratch_types=[
            pltpu.VMEM((SHARD,), jnp.int32),
            pltpu.VMEM((SHARD,), jnp.int32),
            pltpu.VMEM((2 * NP,), jnp.float32),
            pltpu.VMEM((NP,), jnp.int32),
            pltpu.VMEM((SHARD,), jnp.int32),
        ],
    )
    return k(src, dst)


def _sc_layer_body(y_hbm, pk_hbm, agg_hbm, yslab, ypk, aslab, ebuf, claim,
                   mbuf):
    wid = _wid()
    fbase = wid * FPT * NP
    pltpu.sync_copy(y_hbm.at[pl.ds(fbase, FPT * NP)], yslab)
    _zero_ref(aslab, FPT * NP)
    lane = lax.iota(jnp.int32, LANES)
    zero16 = jnp.zeros((LANES,), jnp.int32)
    one16 = jnp.ones((LANES,), jnp.int32)
    himask = jnp.full((LANES,), -65536, jnp.int32)
    rnd = jnp.full((LANES,), 0x8000, jnp.int32)

    def packrow(i, _):
        for p in range(FPT // 2):
            a = yslab[pl.ds((2 * p) * NP + i * LANES, LANES)]
            b = yslab[pl.ds((2 * p + 1) * NP + i * LANES, LANES)]
            au = plsc.bitcast(a, jnp.int32)
            bu = plsc.bitcast(b, jnp.int32)
            lo = lax.shift_right_logical(au + rnd, 16)
            hi = (bu + rnd) & himask
            ypk[pl.ds(p * NP + i * LANES, LANES)] = lo | hi
        return 0

    lax.fori_loop(0, NP // LANES, packrow, 0)

    def unpack(w):
        lof = plsc.bitcast(lax.shift_left(w, 16), jnp.float32)
        hif = plsc.bitcast(w & himask, jnp.float32)
        return lof, hif

    def chunk(g, _):
        ebase = g * CH
        pltpu.sync_copy(pk_hbm.at[pl.ds(ebase, CH)], ebuf)

        @plsc.parallel_loop(0, VPC, unroll=UNROLL)
        def hot(v):
            pk = ebuf[pl.ds(v * LANES, LANES)]
            s = pk & 0x3FFF
            d = lax.shift_right_logical(pk, 14)
            plsc.store_scatter(claim, (d,), lane)
            won = plsc.load_gather(claim, (d,))
            safe = won == lane
            for p in range(FPT // 2):
                w = plsc.load_gather(ypk, (s + p * NP,))
                lof, hif = unpack(w)
                plsc.addupdate_scatter(aslab, (d + (2 * p) * NP,), lof,
                                       mask=safe)
                plsc.addupdate_scatter(aslab, (d + (2 * p + 1) * NP,), hif,
                                       mask=safe)
            mbuf[pl.ds(v * LANES, LANES)] = jnp.where(safe, zero16, one16)

        def grp(gi, _):
            acc = mbuf[pl.ds(gi * G * LANES, LANES)]
            for k in range(1, G):
                acc = acc | mbuf[pl.ds((gi * G + k) * LANES, LANES)]

            @pl.when(jnp.any(acc != 0))
            def _():
                def per_vreg(vv, _):
                    v = gi * G + vv
                    lose = mbuf[pl.ds(v * LANES, LANES)] != 0

                    @pl.when(jnp.any(lose))
                    def _():
                        pk = ebuf[pl.ds(v * LANES, LANES)]
                        s = pk & 0x3FFF
                        d = lax.shift_right_logical(pk, 14)

                        def cond(r):
                            return jnp.any(r)

                        def tail(r):
                            plsc.store_scatter(claim, (d,), lane, mask=r)
                            w2 = plsc.load_gather(claim, (d,), mask=r)
                            ok = jnp.logical_and(r, w2 == lane)
                            for p in range(FPT // 2):
                                w = plsc.load_gather(ypk, (s + p * NP,),
                                                     mask=ok)
                                lof, hif = unpack(w)
                                plsc.addupdate_scatter(
                                    aslab, (d + (2 * p) * NP,), lof, mask=ok)
                                plsc.addupdate_scatter(
                                    aslab, (d + (2 * p + 1) * NP,), hif,
                                    mask=ok)
                            return jnp.logical_and(r, jnp.logical_not(ok))

                        lax.while_loop(cond, tail, lose)

                    return 0

                lax.fori_loop(0, G, per_vreg, 0)

            return 0

        lax.fori_loop(0, VPC // G, grp, 0)
        return 0

    lax.fori_loop(0, NCH, chunk, 0)
    pltpu.sync_copy(aslab, agg_hbm.at[pl.ds(fbase, FPT * NP)])


def _sc_layer(y_flat, packed):
    mesh = plsc.VectorSubcoreMesh(core_axis_name="c", subcore_axis_name="s")
    k = pl.kernel(
        _sc_layer_body,
        out_type=jax.ShapeDtypeStruct((D * NP,), jnp.float32),
        mesh=mesh,
        compiler_params=pltpu.CompilerParams(needs_layout_passes=False),
        scratch_types=[
            pltpu.VMEM((FPT * NP,), jnp.float32),
            pltpu.VMEM((FPT // 2 * NP,), jnp.int32),
            pltpu.VMEM((FPT * NP,), jnp.float32),
            pltpu.VMEM((CH,), jnp.int32),
            pltpu.VMEM((NP,), jnp.int32),
            pltpu.VMEM((CH,), jnp.int32),
        ],
    )
    return k(y_flat, packed)


def _tc1_body(h_ref, w1_ref, degp_ref, y_ref, on_ref, in_ref):
    deg = jnp.sum(degp_ref[...], axis=1)
    onorm = lax.rsqrt(jnp.maximum(deg[0:1, :], 1.0))
    inorm = lax.rsqrt(jnp.maximum(deg[1:2, :], 1.0))
    y = lax.dot_general(w1_ref[...], h_ref[...], (((0,), (1,)), ((), ())),
                        preferred_element_type=jnp.float32)
    y_ref[...] = y * onorm
    on_ref[...] = onorm
    in_ref[...] = inorm


def _tc1(h_pad, w1, degp):
    return pl.pallas_call(
        _tc1_body,
        grid=(GRID,),
        in_specs=[
            pl.BlockSpec((BN, D), lambda i: (i, 0)),
            pl.BlockSpec((D, D), lambda i: (0, 0)),
            pl.BlockSpec((2, NW, BN), lambda i: (0, 0, i)),
        ],
        out_specs=[
            pl.BlockSpec((D, BN), lambda i: (0, i)),
            pl.BlockSpec((1, BN), lambda i: (0, i)),
            pl.BlockSpec((1, BN), lambda i: (0, i)),
        ],
        out_shape=[
            jax.ShapeDtypeStruct((D, NP), jnp.float32),
            jax.ShapeDtypeStruct((1, NP), jnp.float32),
            jax.ShapeDtypeStruct((1, NP), jnp.float32),
        ],
    )(h_pad, w1, degp)


def _tc_mid_body(agg_ref, in_ref, on_ref, b_ref, w_ref, y_ref):
    hprev = jnp.maximum(agg_ref[...] * in_ref[...] + b_ref[...], 0.0)
    y = lax.dot_general(w_ref[...], hprev, (((0,), (0,)), ((), ())),
                        preferred_element_type=jnp.float32)
    y_ref[...] = y * on_ref[...]


def _tc_mid(agg, inorm, onorm, b_prev, w):
    return pl.pallas_call(
        _tc_mid_body,
        grid=(GRID,),
        in_specs=[
            pl.BlockSpec((D, BN), lambda i: (0, i)),
            pl.BlockSpec((1, BN), lambda i: (0, i)),
            pl.BlockSpec((1, BN), lambda i: (0, i)),
            pl.BlockSpec((D, 1), lambda i: (0, 0)),
            pl.BlockSpec((D, D), lambda i: (0, 0)),
        ],
        out_specs=pl.BlockSpec((D, BN), lambda i: (0, i)),
        out_shape=jax.ShapeDtypeStruct((D, NP), jnp.float32),
    )(agg, inorm, onorm, b_prev, w)


def _tc_fin_body(agg_ref, in_ref, b_ref, o_ref):
    i = pl.program_id(0)
    h3 = jnp.maximum(agg_ref[...] * in_ref[...] + b_ref[...], 0.0)
    col = lax.broadcasted_iota(jnp.int32, (1, BN), 1) + i * BN
    h3 = jnp.where(col < N, h3, 0.0)
    part = jnp.sum(h3, axis=1)

    @pl.when(i == 0)
    def _():
        o_ref[...] = jnp.zeros_like(o_ref)

    o_ref[...] += part[None, :]

    @pl.when(i == GRID - 1)
    def _():
        o_ref[...] *= jnp.float32(1.0 / N)


def _tc_fin(agg3, inorm, b3):
    return pl.pallas_call(
        _tc_fin_body,
        grid=(GRID,),
        in_specs=[
            pl.BlockSpec((D, BN), lambda i: (0, i)),
            pl.BlockSpec((1, BN), lambda i: (0, i)),
            pl.BlockSpec((D, 1), lambda i: (0, 0)),
        ],
        out_specs=pl.BlockSpec((1, D), lambda i: (0, 0)),
        out_shape=jax.ShapeDtypeStruct((1, D), jnp.float32),
    )(agg3, inorm, b3)


def kernel(h, edge_index, W1, b1, W2, b2, W3, b3):
    src = edge_index[0]
    dst = edge_index[1]
    h_pad = jnp.pad(h, ((0, NP - N), (0, 0)))

    degp_flat, packed = _sc_deg(src, dst)
    degp = degp_flat.reshape(2, NW, NP)
    y1, onorm, inorm = _tc1(h_pad, W1, degp)

    agg1 = _sc_layer(y1.reshape(-1), packed).reshape(D, NP)
    y2 = _tc_mid(agg1, inorm, onorm, b1.reshape(D, 1), W2)

    agg2 = _sc_layer(y2.reshape(-1), packed).reshape(D, NP)
    y3 = _tc_mid(agg2, inorm, onorm, b2.reshape(D, 1), W3)

    agg3 = _sc_layer(y3.reshape(-1), packed).reshape(D, NP)
    return _tc_fin(agg3, inorm, b3.reshape(D, 1))


# speedup vs baseline: 8.3547x; 2.5276x over previous
import jax
import jax.numpy as jnp
import numpy as np
from jax import lax
from jax.experimental import pallas as pl
from jax.experimental.pallas import tpu as pltpu
from jax.experimental.pallas import tpu_sc as plsc

N = 10000
NP = 10240
E = 320000
D = 128
NW = 32
FPT = D // NW
SHARD = E // NW
RCAP = SHARD + 272
BN = 1280
GRID = NP // BN
LANES = 16
UNROLL = 8
PAD_S = NP - 1
PAD_D = NP - LANES
PADPK = PAD_S | (PAD_D << 14)
HIMASK = -65536


def _wid():
    return lax.axis_index("s") * 2 + lax.axis_index("c")


def _zero_ref(ref, nwords):
    z = jnp.zeros((LANES,), jnp.float32)

    def body(i, _):
        ref[pl.ds(i * LANES, LANES)] = z
        return 0

    lax.fori_loop(0, nwords // LANES, body, 0)


def _sc_deg_body(src_hbm, dst_hbm, degp_hbm, packed_hbm, sbuf, dbuf, hist,
                 claim, pbuf):
    wid = _wid()
    base = wid * SHARD
    pltpu.sync_copy(src_hbm.at[pl.ds(base, SHARD)], sbuf)
    pltpu.sync_copy(dst_hbm.at[pl.ds(base, SHARD)], dbuf)
    _zero_ref(hist, 2 * NP)
    lane = lax.iota(jnp.int32, LANES)
    ones = jnp.ones((LANES,), jnp.float32)

    def body(v, _):
        s = sbuf[pl.ds(v * LANES, LANES)]
        plsc.store_scatter(claim, (s,), lane)
        won = plsc.load_gather(claim, (s,))
        safe = won == lane
        plsc.addupdate_scatter(hist, (s,), ones, mask=safe)
        rem = jnp.logical_not(safe)

        def cond(r):
            return jnp.any(r)

        def tail(r):
            plsc.store_scatter(claim, (s,), lane, mask=r)
            w2 = plsc.load_gather(claim, (s,), mask=r)
            ok = jnp.logical_and(r, w2 == lane)
            plsc.addupdate_scatter(hist, (s,), ones, mask=ok)
            return jnp.logical_and(r, jnp.logical_not(ok))

        lax.while_loop(cond, tail, rem)

        d = dbuf[pl.ds(v * LANES, LANES)]
        pbuf[pl.ds(v * LANES, LANES)] = s | (d << 14)
        dt = d + NP
        plsc.store_scatter(claim, (d,), lane)
        wond = plsc.load_gather(claim, (d,))
        safed = wond == lane
        plsc.addupdate_scatter(hist, (dt,), ones, mask=safed)
        remd = jnp.logical_not(safed)

        def taild(r):
            plsc.store_scatter(claim, (d,), lane, mask=r)
            w2 = plsc.load_gather(claim, (d,), mask=r)
            ok = jnp.logical_and(r, w2 == lane)
            plsc.addupdate_scatter(hist, (dt,), ones, mask=ok)
            return jnp.logical_and(r, jnp.logical_not(ok))

        lax.while_loop(cond, taild, remd)
        return 0

    lax.fori_loop(0, SHARD // LANES, body, 0)
    pltpu.sync_copy(hist.at[pl.ds(0, NP)], degp_hbm.at[pl.ds(wid * NP, NP)])
    pltpu.sync_copy(hist.at[pl.ds(NP, NP)],
                    degp_hbm.at[pl.ds(NW * NP + wid * NP, NP)])
    pltpu.sync_copy(pbuf, packed_hbm.at[pl.ds(base, SHARD)])


def _sc_deg(src, dst):
    mesh = plsc.VectorSubcoreMesh(core_axis_name="c", subcore_axis_name="s")
    k = pl.kernel(
        _sc_deg_body,
        out_type=[
            jax.ShapeDtypeStruct((2 * NW * NP,), jnp.float32),
            jax.ShapeDtypeStruct((E,), jnp.int32),
        ],
        mesh=mesh,
        compiler_params=pltpu.CompilerParams(needs_layout_passes=False),
        scratch_types=[
            pltpu.VMEM((SHARD,), jnp.int32),
            pltpu.VMEM((SHARD,), jnp.int32),
            pltpu.VMEM((2 * NP,), jnp.float32),
            pltpu.VMEM((NP,), jnp.int32),
            pltpu.VMEM((SHARD,), jnp.int32),
        ],
    )
    return k(src, dst)


def _sc_part_body(pk_hbm, stream_hbm, meta_hbm, ebuf, cbuf, mvbuf):
    wid = _wid()
    base = wid * SHARD
    pltpu.sync_copy(pk_hbm.at[pl.ds(base, SHARD)], ebuf)
    lane = lax.iota(jnp.int32, LANES)
    padvec = jnp.full((LANES,), PADPK, jnp.int32)

    def pf(i, _):
        cbuf[pl.ds(i * LANES, LANES)] = padvec
        return 0

    lax.fori_loop(0, RCAP // LANES, pf, 0)

    def cnt_body(v, cntv):
        pk = ebuf[pl.ds(v * LANES, LANES)]
        cls = lax.shift_right_logical(pk, 14) & 15
        for c in range(LANES):
            pc = plsc.all_reduce_population_count(cls == c)
            cntv = cntv + jnp.where(lane == c, pc, 0)
        return cntv

    cntv = lax.fori_loop(0, SHARD // LANES, cnt_body,
                         jnp.zeros((LANES,), jnp.int32))
    cnt16 = (cntv + 15) & -16
    pref = plsc.cumsum(cnt16) - cnt16 + lane

    def scal(vec, c):
        return jnp.sum(jnp.where(lane == c, vec, jnp.int32(0)))

    pos0 = tuple(scal(pref, c) for c in range(LANES))

    def part_body(v, pos):
        pk = ebuf[pl.ds(v * LANES, LANES)]
        cls = lax.shift_right_logical(pk, 14) & 15
        new = []
        for c in range(LANES):
            m = cls == c
            plsc.store_compressed(cbuf.at[pl.ds(pos[c], LANES)], pk, mask=m)
            new.append(pos[c] + jnp.sum(m.astype(jnp.int32)))
        return tuple(new)

    lax.fori_loop(0, SHARD // LANES, part_body, pos0)

    pltpu.sync_copy(cbuf, stream_hbm.at[pl.ds(wid * RCAP, RCAP)])
    mvbuf[pl.ds(0, LANES)] = pref
    mvbuf[pl.ds(LANES, LANES)] = cntv
    pltpu.sync_copy(mvbuf, meta_hbm.at[pl.ds(wid * 2 * LANES, 2 * LANES)])


def _sc_part(packed):
    mesh = plsc.VectorSubcoreMesh(core_axis_name="c", subcore_axis_name="s")
    k = pl.kernel(
        _sc_part_body,
        out_type=[
            jax.ShapeDtypeStruct((NW * RCAP,), jnp.int32),
            jax.ShapeDtypeStruct((NW * 2 * LANES,), jnp.int32),
        ],
        mesh=mesh,
        compiler_params=pltpu.CompilerParams(needs_layout_passes=False),
        scratch_types=[
            pltpu.VMEM((SHARD,), jnp.int32),
            pltpu.VMEM((RCAP,), jnp.int32),
            pltpu.VMEM((2 * LANES,), jnp.int32),
        ],
    )
    return k(packed)


def _sc_layer_body(yp_hbm, stream_hbm, meta_hbm, agg_hbm, ypk, aslab, ebuf,
                   mvbuf):
    wid = _wid()
    pltpu.sync_copy(yp_hbm.at[pl.ds(wid * 2 * NP, 2 * NP)], ypk)
    _zero_ref(aslab, FPT * NP)
    himask = jnp.full((LANES,), HIMASK, jnp.int32)

    def region(w2, _):
        pltpu.sync_copy(meta_hbm.at[pl.ds(w2 * 2 * LANES, 2 * LANES)], mvbuf)
        pltpu.sync_copy(stream_hbm.at[pl.ds(w2 * RCAP, RCAP)], ebuf)
        lstart = mvbuf[pl.ds(0, LANES)]
        cntv = mvbuf[pl.ds(LANES, LANES)]
        maxc = jnp.max(cntv)

        @plsc.parallel_loop(0, maxc, unroll=UNROLL)
        def hot(j):
            valid = j < cntv
            pk = plsc.load_gather(ebuf, (lstart + j,), mask=valid)
            s = pk & 0x3FFF
            d = lax.shift_right_logical(pk, 14)
            for p in range(FPT // 2):
                w = plsc.load_gather(ypk, (s + p * NP,), mask=valid)
                lof = plsc.bitcast(lax.shift_left(w, 16), jnp.float32)
                hif = plsc.bitcast(w & himask, jnp.float32)
                plsc.addupdate_scatter(aslab, (d + (2 * p) * NP,), lof,
                                       mask=valid)
                plsc.addupdate_scatter(aslab, (d + (2 * p + 1) * NP,), hif,
                                       mask=valid)

        return 0

    lax.fori_loop(0, NW, region, 0)
    pltpu.sync_copy(aslab, agg_hbm.at[pl.ds(wid * FPT * NP, FPT * NP)])


def _sc_layer(yp_flat, stream, meta):
    mesh = plsc.VectorSubcoreMesh(core_axis_name="c", subcore_axis_name="s")
    k = pl.kernel(
        _sc_layer_body,
        out_type=jax.ShapeDtypeStruct((D * NP,), jnp.float32),
        mesh=mesh,
        compiler_params=pltpu.CompilerParams(needs_layout_passes=False),
        scratch_types=[
            pltpu.VMEM((2 * NP,), jnp.int32),
            pltpu.VMEM((FPT * NP,), jnp.float32),
            pltpu.VMEM((RCAP,), jnp.int32),
            pltpu.VMEM((2 * LANES,), jnp.int32),
        ],
    )
    return k(yp_flat, stream, meta)


def _pack_pairs(y):
    au = lax.bitcast_convert_type(y[0:64, :], jnp.int32)
    bu = lax.bitcast_convert_type(y[64:128, :], jnp.int32)
    rnd = jnp.int32(0x8000)
    lo = lax.shift_right_logical(au + rnd, 16)
    hi = (bu + rnd) & jnp.int32(HIMASK)
    return lo | hi


def _tc1_body(h_ref, w1_ref, degp_ref, y_ref, on_ref, in_ref):
    deg = jnp.sum(degp_ref[...], axis=1)
    onorm = lax.rsqrt(jnp.maximum(deg[0:1, :], 1.0))
    inorm = lax.rsqrt(jnp.maximum(deg[1:2, :], 1.0))
    y = lax.dot_general(w1_ref[...], h_ref[...], (((0,), (1,)), ((), ())),
                        preferred_element_type=jnp.float32)
    y_ref[...] = _pack_pairs(y * onorm)
    on_ref[...] = onorm
    in_ref[...] = inorm


def _tc1(h_pad, w1p, degp):
    return pl.pallas_call(
        _tc1_body,
        grid=(GRID,),
        in_specs=[
            pl.BlockSpec((BN, D), lambda i: (i, 0)),
            pl.BlockSpec((D, D), lambda i: (0, 0)),
            pl.BlockSpec((2, NW, BN), lambda i: (0, 0, i)),
        ],
        out_specs=[
            pl.BlockSpec((D // 2, BN), lambda i: (0, i)),
            pl.BlockSpec((1, BN), lambda i: (0, i)),
            pl.BlockSpec((1, BN), lambda i: (0, i)),
        ],
        out_shape=[
            jax.ShapeDtypeStruct((D // 2, NP), jnp.int32),
            jax.ShapeDtypeStruct((1, NP), jnp.float32),
            jax.ShapeDtypeStruct((1, NP), jnp.float32),
        ],
    )(h_pad, w1p, degp)


def _tc_mid_body(agg_ref, in_ref, on_ref, b_ref, w_ref, y_ref):
    hprev = jnp.maximum(agg_ref[...] * in_ref[...] + b_ref[...], 0.0)
    y = lax.dot_general(w_ref[...], hprev, (((0,), (0,)), ((), ())),
                        preferred_element_type=jnp.float32)
    y_ref[...] = _pack_pairs(y * on_ref[...])


def _tc_mid(agg, inorm, onorm, b_prev, wp):
    return pl.pallas_call(
        _tc_mid_body,
        grid=(GRID,),
        in_specs=[
            pl.BlockSpec((D, BN), lambda i: (0, i)),
            pl.BlockSpec((1, BN), lambda i: (0, i)),
            pl.BlockSpec((1, BN), lambda i: (0, i)),
            pl.BlockSpec((D, 1), lambda i: (0, 0)),
            pl.BlockSpec((D, D), lambda i: (0, 0)),
        ],
        out_specs=pl.BlockSpec((D // 2, BN), lambda i: (0, i)),
        out_shape=jax.ShapeDtypeStruct((D // 2, NP), jnp.int32),
    )(agg, inorm, onorm, b_prev, wp)


def _tc_fin_body(agg_ref, in_ref, b_ref, o_ref):
    i = pl.program_id(0)
    h3 = jnp.maximum(agg_ref[...] * in_ref[...] + b_ref[...], 0.0)
    col = lax.broadcasted_iota(jnp.int32, (1, BN), 1) + i * BN
    h3 = jnp.where(col < N, h3, 0.0)
    part = jnp.sum(h3, axis=1)

    @pl.when(i == 0)
    def _():
        o_ref[...] = jnp.zeros_like(o_ref)

    o_ref[...] += part[None, :]

    @pl.when(i == GRID - 1)
    def _():
        o_ref[...] *= jnp.float32(1.0 / N)


def _tc_fin(agg3, inorm, b3):
    return pl.pallas_call(
        _tc_fin_body,
        grid=(GRID,),
        in_specs=[
            pl.BlockSpec((D, BN), lambda i: (0, i)),
            pl.BlockSpec((1, BN), lambda i: (0, i)),
            pl.BlockSpec((D, 1), lambda i: (0, 0)),
        ],
        out_specs=pl.BlockSpec((1, D), lambda i: (0, 0)),
        out_shape=jax.ShapeDtypeStruct((1, D), jnp.float32),
    )(agg3, inorm, b3)


_PERM = np.concatenate([np.arange(0, D, 2), np.arange(1, D, 2)])


def kernel(h, edge_index, W1, b1, W2, b2, W3, b3):
    src = edge_index[0]
    dst = edge_index[1]
    h_pad = jnp.pad(h, ((0, NP - N), (0, 0)))
    perm = jnp.asarray(_PERM)
    w1p = W1[:, perm]
    w2p = W2[:, perm]
    w3p = W3[:, perm]

    degp_flat, packed = _sc_deg(src, dst)
    stream, meta = _sc_part(packed)
    degp = degp_flat.reshape(2, NW, NP)
    y1p, onorm, inorm = _tc1(h_pad, w1p, degp)

    agg1 = _sc_layer(y1p.reshape(-1), stream, meta).reshape(D, NP)
    y2p = _tc_mid(agg1, inorm, onorm, b1.reshape(D, 1), w2p)

    agg2 = _sc_layer(y2p.reshape(-1), stream, meta).reshape(D, NP)
    y3p = _tc_mid(agg2, inorm, onorm, b2.reshape(D, 1), w3p)

    agg3 = _sc_layer(y3p.reshape(-1), stream, meta).reshape(D, NP)
    return _tc_fin(agg3, inorm, b3.reshape(D, 1))


# speedup vs baseline: 10.4304x; 1.2484x over previous
import jax
import jax.numpy as jnp
import numpy as np
from jax import lax
from jax.experimental import pallas as pl
from jax.experimental.pallas import tpu as pltpu
from jax.experimental.pallas import tpu_sc as plsc

N = 10000
NP = 10240
E = 320000
D = 128
NW = 32
FPT = D // NW
SHARD = E // NW
RCAP = SHARD + 272
BN = 1280
GRID = NP // BN
LANES = 16
UNROLL = 8
PAD_S = NP - 1
PAD_D = NP - LANES
PADPK = PAD_S | (PAD_D << 14)
HIMASK = -65536


def _wid():
    return lax.axis_index("s") * 2 + lax.axis_index("c")


def _zero_ref(ref, nwords):
    z = jnp.zeros((LANES,), jnp.float32)

    def body(i, _):
        ref[pl.ds(i * LANES, LANES)] = z
        return 0

    lax.fori_loop(0, nwords // LANES, body, 0)


def _sc_deg_body(src_hbm, dst_hbm, degp_hbm, packed_hbm, sbuf, dbuf, hist,
                 claim, pbuf):
    wid = _wid()
    base = wid * SHARD
    pltpu.sync_copy(src_hbm.at[pl.ds(base, SHARD)], sbuf)
    pltpu.sync_copy(dst_hbm.at[pl.ds(base, SHARD)], dbuf)
    _zero_ref(hist, 2 * NP)
    lane = lax.iota(jnp.int32, LANES)
    ones = jnp.ones((LANES,), jnp.float32)

    def body(v, _):
        s = sbuf[pl.ds(v * LANES, LANES)]
        plsc.store_scatter(claim, (s,), lane)
        won = plsc.load_gather(claim, (s,))
        safe = won == lane
        plsc.addupdate_scatter(hist, (s,), ones, mask=safe)
        rem = jnp.logical_not(safe)

        def cond(r):
            return jnp.any(r)

        def tail(r):
            plsc.store_scatter(claim, (s,), lane, mask=r)
            w2 = plsc.load_gather(claim, (s,), mask=r)
            ok = jnp.logical_and(r, w2 == lane)
            plsc.addupdate_scatter(hist, (s,), ones, mask=ok)
            return jnp.logical_and(r, jnp.logical_not(ok))

        lax.while_loop(cond, tail, rem)

        d = dbuf[pl.ds(v * LANES, LANES)]
        pbuf[pl.ds(v * LANES, LANES)] = s | (d << 14)
        dt = d + NP
        plsc.store_scatter(claim, (d,), lane)
        wond = plsc.load_gather(claim, (d,))
        safed = wond == lane
        plsc.addupdate_scatter(hist, (dt,), ones, mask=safed)
        remd = jnp.logical_not(safed)

        def taild(r):
            plsc.store_scatter(claim, (d,), lane, mask=r)
            w2 = plsc.load_gather(claim, (d,), mask=r)
            ok = jnp.logical_and(r, w2 == lane)
            plsc.addupdate_scatter(hist, (dt,), ones, mask=ok)
            return jnp.logical_and(r, jnp.logical_not(ok))

        lax.while_loop(cond, taild, remd)
        return 0

    lax.fori_loop(0, SHARD // LANES, body, 0)
    pltpu.sync_copy(hist.at[pl.ds(0, NP)], degp_hbm.at[pl.ds(wid * NP, NP)])
    pltpu.sync_copy(hist.at[pl.ds(NP, NP)],
                    degp_hbm.at[pl.ds(NW * NP + wid * NP, NP)])
    pltpu.sync_copy(pbuf, packed_hbm.at[pl.ds(base, SHARD)])


def _sc_deg(src, dst):
    mesh = plsc.VectorSubcoreMesh(core_axis_name="c", subcore_axis_name="s")
    k = pl.kernel(
        _sc_deg_body,
        out_type=[
            jax.ShapeDtypeStruct((2 * NW * NP,), jnp.float32),
            jax.ShapeDtypeStruct((E,), jnp.int32),
        ],
        mesh=mesh,
        compiler_params=pltpu.CompilerParams(needs_layout_passes=False),
        scratch_types=[
            pltpu.VMEM((SHARD,), jnp.int32),
            pltpu.VMEM((SHARD,), jnp.int32),
            pltpu.VMEM((2 * NP,), jnp.float32),
            pltpu.VMEM((NP,), jnp.int32),
            pltpu.VMEM((SHARD,), jnp.int32),
        ],
    )
    return k(src, dst)


def _sc_part_body(pk_hbm, stream_hbm, meta_hbm, ebuf, cbuf, mvbuf):
    wid = _wid()
    base = wid * SHARD
    pltpu.sync_copy(pk_hbm.at[pl.ds(base, SHARD)], ebuf)
    lane = lax.iota(jnp.int32, LANES)
    padvec = jnp.full((LANES,), PADPK, jnp.int32)

    def pf(i, _):
        cbuf[pl.ds(i * LANES, LANES)] = padvec
        return 0

    lax.fori_loop(0, RCAP // LANES, pf, 0)

    def cnt_body(v, cntv):
        pk = ebuf[pl.ds(v * LANES, LANES)]
        cls = lax.shift_right_logical(pk, 14) & 15
        for c in range(LANES):
            pc = plsc.all_reduce_population_count(cls == c)
            cntv = cntv + jnp.where(lane == c, pc, 0)
        return cntv

    cntv = lax.fori_loop(0, SHARD // LANES, cnt_body,
                         jnp.zeros((LANES,), jnp.int32))
    cnt16 = (cntv + 15) & -16
    pref = plsc.cumsum(cnt16) - cnt16 + lane

    def scal(vec, c):
        return jnp.sum(jnp.where(lane == c, vec, jnp.int32(0)))

    pos0 = tuple(scal(pref, c) for c in range(LANES))

    def part_body(v, pos):
        pk = ebuf[pl.ds(v * LANES, LANES)]
        cls = lax.shift_right_logical(pk, 14) & 15
        new = []
        for c in range(LANES):
            m = cls == c
            plsc.store_compressed(cbuf.at[pl.ds(pos[c], LANES)], pk, mask=m)
            new.append(pos[c] + jnp.sum(m.astype(jnp.int32)))
        return tuple(new)

    lax.fori_loop(0, SHARD // LANES, part_body, pos0)

    pltpu.sync_copy(cbuf, stream_hbm.at[pl.ds(wid * RCAP, RCAP)])
    mvbuf[pl.ds(0, LANES)] = pref
    mvbuf[pl.ds(LANES, LANES)] = cntv
    pltpu.sync_copy(mvbuf, meta_hbm.at[pl.ds(wid * 2 * LANES, 2 * LANES)])


def _sc_part(packed):
    mesh = plsc.VectorSubcoreMesh(core_axis_name="c", subcore_axis_name="s")
    k = pl.kernel(
        _sc_part_body,
        out_type=[
            jax.ShapeDtypeStruct((NW * RCAP,), jnp.int32),
            jax.ShapeDtypeStruct((NW * 2 * LANES,), jnp.int32),
        ],
        mesh=mesh,
        compiler_params=pltpu.CompilerParams(needs_layout_passes=False),
        scratch_types=[
            pltpu.VMEM((SHARD,), jnp.int32),
            pltpu.VMEM((RCAP,), jnp.int32),
            pltpu.VMEM((2 * LANES,), jnp.int32),
        ],
    )
    return k(packed)


def _sc_layer_body(yp_hbm, stream_hbm, meta_hbm, agg_hbm, ypk, aslab, ebuf0,
                   ebuf1, mall, sem0, sem1):
    wid = _wid()
    pltpu.sync_copy(meta_hbm, mall)
    pltpu.sync_copy(yp_hbm.at[pl.ds(wid * 2 * NP, 2 * NP)], ypk)
    _zero_ref(aslab, FPT * NP)
    himask = jnp.full((LANES,), HIMASK, jnp.int32)
    ebufs = (ebuf0, ebuf1)
    sems = (sem0, sem1)

    pltpu.async_copy(stream_hbm.at[pl.ds(0, RCAP)], ebuf0, sem0)

    def process(w2, ebuf):
        lstart = mall[pl.ds(w2 * 2 * LANES, LANES)]
        cntv = mall[pl.ds(w2 * 2 * LANES + LANES, LANES)]
        maxc = jnp.max(cntv)

        @plsc.parallel_loop(0, maxc, unroll=UNROLL)
        def hot(j):
            valid = j < cntv
            pk = plsc.load_gather(ebuf, (lstart + j,), mask=valid)
            s = pk & 0x3FFF
            d = lax.shift_right_logical(pk, 14)
            for p in range(FPT // 2):
                w = plsc.load_gather(ypk, (s + p * NP,), mask=valid)
                lof = plsc.bitcast(lax.shift_left(w, 16), jnp.float32)
                hif = plsc.bitcast(w & himask, jnp.float32)
                plsc.addupdate_scatter(aslab, (d + (2 * p) * NP,), lof,
                                       mask=valid)
                plsc.addupdate_scatter(aslab, (d + (2 * p + 1) * NP,), hif,
                                       mask=valid)

    def pair(pi, _):
        for sl in range(2):
            w2 = pi * 2 + sl
            nxt = w2 + 1

            @pl.when(nxt < NW)
            def _():
                pltpu.async_copy(stream_hbm.at[pl.ds(nxt * RCAP, RCAP)],
                                 ebufs[1 - sl], sems[1 - sl])

            pltpu.make_async_copy(stream_hbm.at[pl.ds(0, RCAP)], ebufs[sl],
                                  sems[sl]).wait()
            process(w2, ebufs[sl])
        return 0

    lax.fori_loop(0, NW // 2, pair, 0)
    pltpu.sync_copy(aslab, agg_hbm.at[pl.ds(wid * FPT * NP, FPT * NP)])


def _sc_layer(yp_flat, stream, meta):
    mesh = plsc.VectorSubcoreMesh(core_axis_name="c", subcore_axis_name="s")
    k = pl.kernel(
        _sc_layer_body,
        out_type=jax.ShapeDtypeStruct((D * NP,), jnp.float32),
        mesh=mesh,
        compiler_params=pltpu.CompilerParams(needs_layout_passes=False),
        scratch_types=[
            pltpu.VMEM((2 * NP,), jnp.int32),
            pltpu.VMEM((FPT * NP,), jnp.float32),
            pltpu.VMEM((RCAP,), jnp.int32),
            pltpu.VMEM((RCAP,), jnp.int32),
            pltpu.VMEM((NW * 2 * LANES,), jnp.int32),
            pltpu.SemaphoreType.DMA,
            pltpu.SemaphoreType.DMA,
        ],
    )
    return k(yp_flat, stream, meta)


def _pack_pairs(y):
    au = lax.bitcast_convert_type(y[0:64, :], jnp.int32)
    bu = lax.bitcast_convert_type(y[64:128, :], jnp.int32)
    rnd = jnp.int32(0x8000)
    lo = lax.shift_right_logical(au + rnd, 16)
    hi = (bu + rnd) & jnp.int32(HIMASK)
    return lo | hi


def _tc1_body(h_ref, w1_ref, degp_ref, y_ref, on_ref, in_ref):
    deg = jnp.sum(degp_ref[...], axis=1)
    onorm = lax.rsqrt(jnp.maximum(deg[0:1, :], 1.0))
    inorm = lax.rsqrt(jnp.maximum(deg[1:2, :], 1.0))
    y = lax.dot_general(w1_ref[...], h_ref[...], (((0,), (1,)), ((), ())),
                        preferred_element_type=jnp.float32)
    y_ref[...] = _pack_pairs(y * onorm)
    on_ref[...] = onorm
    in_ref[...] = inorm


def _tc1(h_pad, w1p, degp):
    return pl.pallas_call(
        _tc1_body,
        grid=(GRID,),
        in_specs=[
            pl.BlockSpec((BN, D), lambda i: (i, 0)),
            pl.BlockSpec((D, D), lambda i: (0, 0)),
            pl.BlockSpec((2, NW, BN), lambda i: (0, 0, i)),
        ],
        out_specs=[
            pl.BlockSpec((D // 2, BN), lambda i: (0, i)),
            pl.BlockSpec((1, BN), lambda i: (0, i)),
            pl.BlockSpec((1, BN), lambda i: (0, i)),
        ],
        out_shape=[
            jax.ShapeDtypeStruct((D // 2, NP), jnp.int32),
            jax.ShapeDtypeStruct((1, NP), jnp.float32),
            jax.ShapeDtypeStruct((1, NP), jnp.float32),
        ],
    )(h_pad, w1p, degp)


def _tc_mid_body(agg_ref, in_ref, on_ref, b_ref, w_ref, y_ref):
    hprev = jnp.maximum(agg_ref[...] * in_ref[...] + b_ref[...], 0.0)
    y = lax.dot_general(w_ref[...], hprev, (((0,), (0,)), ((), ())),
                        preferred_element_type=jnp.float32)
    y_ref[...] = _pack_pairs(y * on_ref[...])


def _tc_mid(agg, inorm, onorm, b_prev, wp):
    return pl.pallas_call(
        _tc_mid_body,
        grid=(GRID,),
        in_specs=[
            pl.BlockSpec((D, BN), lambda i: (0, i)),
            pl.BlockSpec((1, BN), lambda i: (0, i)),
            pl.BlockSpec((1, BN), lambda i: (0, i)),
            pl.BlockSpec((D, 1), lambda i: (0, 0)),
            pl.BlockSpec((D, D), lambda i: (0, 0)),
        ],
        out_specs=pl.BlockSpec((D // 2, BN), lambda i: (0, i)),
        out_shape=jax.ShapeDtypeStruct((D // 2, NP), jnp.int32),
    )(agg, inorm, onorm, b_prev, wp)


def _tc_fin_body(agg_ref, in_ref, b_ref, o_ref):
    i = pl.program_id(0)
    h3 = jnp.maximum(agg_ref[...] * in_ref[...] + b_ref[...], 0.0)
    col = lax.broadcasted_iota(jnp.int32, (1, BN), 1) + i * BN
    h3 = jnp.where(col < N, h3, 0.0)
    part = jnp.sum(h3, axis=1)

    @pl.when(i == 0)
    def _():
        o_ref[...] = jnp.zeros_like(o_ref)

    o_ref[...] += part[None, :]

    @pl.when(i == GRID - 1)
    def _():
        o_ref[...] *= jnp.float32(1.0 / N)


def _tc_fin(agg3, inorm, b3):
    return pl.pallas_call(
        _tc_fin_body,
        grid=(GRID,),
        in_specs=[
            pl.BlockSpec((D, BN), lambda i: (0, i)),
            pl.BlockSpec((1, BN), lambda i: (0, i)),
            pl.BlockSpec((D, 1), lambda i: (0, 0)),
        ],
        out_specs=pl.BlockSpec((1, D), lambda i: (0, 0)),
        out_shape=jax.ShapeDtypeStruct((1, D), jnp.float32),
    )(agg3, inorm, b3)


_PERM = np.concatenate([np.arange(0, D, 2), np.arange(1, D, 2)])


def kernel(h, edge_index, W1, b1, W2, b2, W3, b3):
    src = edge_index[0]
    dst = edge_index[1]
    h_pad = jnp.pad(h, ((0, NP - N), (0, 0)))
    perm = jnp.asarray(_PERM)
    w1p = W1[:, perm]
    w2p = W2[:, perm]
    w3p = W3[:, perm]

    degp_flat, packed = _sc_deg(src, dst)
    stream, meta = _sc_part(packed)
    degp = degp_flat.reshape(2, NW, NP)
    y1p, onorm, inorm = _tc1(h_pad, w1p, degp)

    agg1 = _sc_layer(y1p.reshape(-1), stream, meta).reshape(D, NP)
    y2p = _tc_mid(agg1, inorm, onorm, b1.reshape(D, 1), w2p)

    agg2 = _sc_layer(y2p.reshape(-1), stream, meta).reshape(D, NP)
    y3p = _tc_mid(agg2, inorm, onorm, b2.reshape(D, 1), w3p)

    agg3 = _sc_layer(y3p.reshape(-1), stream, meta).reshape(D, NP)
    return _tc_fin(agg3, inorm, b3.reshape(D, 1))


# speedup vs baseline: 10.9557x; 1.0504x over previous
import jax
import jax.numpy as jnp
import numpy as np
from jax import lax
from jax.experimental import pallas as pl
from jax.experimental.pallas import tpu as pltpu
from jax.experimental.pallas import tpu_sc as plsc

N = 10000
NP = 10240
E = 320000
D = 128
NW = 32
FPT = D // NW
SHARD = E // NW
RCAP = SHARD + 272
BN = 1280
GRID = NP // BN
LANES = 16
UNROLL = 8
PAD_S = NP - 1
PAD_D = NP - LANES
PADPK = PAD_S | (PAD_D << 14)
HIMASK = -65536


def _wid():
    return lax.axis_index("s") * 2 + lax.axis_index("c")


def _zero_ref(ref, nwords):
    z = jnp.zeros((LANES,), jnp.float32)

    @plsc.parallel_loop(0, nwords // LANES, unroll=8)
    def body(i):
        ref[pl.ds(i * LANES, LANES)] = z


def _sc_deg_body(src_hbm, dst_hbm, degp_hbm, stream_hbm, meta_hbm, sbuf, dbuf,
                 hist, claim, pbuf, cbuf, mvbuf):
    wid = _wid()
    base = wid * SHARD
    pltpu.sync_copy(src_hbm.at[pl.ds(base, SHARD)], sbuf)
    pltpu.sync_copy(dst_hbm.at[pl.ds(base, SHARD)], dbuf)
    _zero_ref(hist, 2 * NP)
    lane = lax.iota(jnp.int32, LANES)
    ones = jnp.ones((LANES,), jnp.float32)

    def body(v, _):
        s = sbuf[pl.ds(v * LANES, LANES)]
        plsc.store_scatter(claim, (s,), lane)
        won = plsc.load_gather(claim, (s,))
        safe = won == lane
        plsc.addupdate_scatter(hist, (s,), ones, mask=safe)
        rem = jnp.logical_not(safe)

        def cond(r):
            return jnp.any(r)

        def tail(r):
            plsc.store_scatter(claim, (s,), lane, mask=r)
            w2 = plsc.load_gather(claim, (s,), mask=r)
            ok = jnp.logical_and(r, w2 == lane)
            plsc.addupdate_scatter(hist, (s,), ones, mask=ok)
            return jnp.logical_and(r, jnp.logical_not(ok))

        lax.while_loop(cond, tail, rem)

        d = dbuf[pl.ds(v * LANES, LANES)]
        pbuf[pl.ds(v * LANES, LANES)] = s | (d << 14)
        dt = d + NP
        plsc.store_scatter(claim, (d,), lane)
        wond = plsc.load_gather(claim, (d,))
        safed = wond == lane
        plsc.addupdate_scatter(hist, (dt,), ones, mask=safed)
        remd = jnp.logical_not(safed)

        def taild(r):
            plsc.store_scatter(claim, (d,), lane, mask=r)
            w2 = plsc.load_gather(claim, (d,), mask=r)
            ok = jnp.logical_and(r, w2 == lane)
            plsc.addupdate_scatter(hist, (dt,), ones, mask=ok)
            return jnp.logical_and(r, jnp.logical_not(ok))

        lax.while_loop(cond, taild, remd)
        return 0

    lax.fori_loop(0, SHARD // LANES, body, 0)
    pltpu.sync_copy(hist.at[pl.ds(0, NP)], degp_hbm.at[pl.ds(wid * NP, NP)])
    pltpu.sync_copy(hist.at[pl.ds(NP, NP)],
                    degp_hbm.at[pl.ds(NW * NP + wid * NP, NP)])

    _part_shard(pbuf, cbuf, mvbuf, lane)
    pltpu.sync_copy(cbuf, stream_hbm.at[pl.ds(wid * RCAP, RCAP)])
    pltpu.sync_copy(mvbuf, meta_hbm.at[pl.ds(wid * 2 * LANES, 2 * LANES)])


def _sc_deg(src, dst):
    mesh = plsc.VectorSubcoreMesh(core_axis_name="c", subcore_axis_name="s")
    k = pl.kernel(
        _sc_deg_body,
        out_type=[
            jax.ShapeDtypeStruct((2 * NW * NP,), jnp.float32),
            jax.ShapeDtypeStruct((NW * RCAP,), jnp.int32),
            jax.ShapeDtypeStruct((NW * 2 * LANES,), jnp.int32),
        ],
        mesh=mesh,
        compiler_params=pltpu.CompilerParams(needs_layout_passes=False),
        scratch_types=[
            pltpu.VMEM((SHARD,), jnp.int32),
            pltpu.VMEM((SHARD,), jnp.int32),
            pltpu.VMEM((2 * NP,), jnp.float32),
            pltpu.VMEM((NP,), jnp.int32),
            pltpu.VMEM((SHARD,), jnp.int32),
            pltpu.VMEM((RCAP,), jnp.int32),
            pltpu.VMEM((2 * LANES,), jnp.int32),
        ],
    )
    return k(src, dst)


def _part_shard(ebuf, cbuf, mvbuf, lane):
    padvec = jnp.full((LANES,), PADPK, jnp.int32)

    def pf(i, _):
        cbuf[pl.ds(i * LANES, LANES)] = padvec
        return 0

    lax.fori_loop(0, RCAP // LANES, pf, 0)

    def cnt_body(v, cntv):
        pk = ebuf[pl.ds(v * LANES, LANES)]
        cls = lax.shift_right_logical(pk, 14) & 15
        for c in range(LANES):
            pc = plsc.all_reduce_population_count(cls == c)
            cntv = cntv + jnp.where(lane == c, pc, 0)
        return cntv

    cntv = lax.fori_loop(0, SHARD // LANES, cnt_body,
                         jnp.zeros((LANES,), jnp.int32))
    cnt16 = (cntv + 15) & -16
    pref = plsc.cumsum(cnt16) - cnt16 + lane

    def scal(vec, c):
        return jnp.sum(jnp.where(lane == c, vec, jnp.int32(0)))

    pos0 = tuple(scal(pref, c) for c in range(LANES))

    def part_body(v, pos):
        pk = ebuf[pl.ds(v * LANES, LANES)]
        cls = lax.shift_right_logical(pk, 14) & 15
        new = []
        for c in range(LANES):
            m = cls == c
            plsc.store_compressed(cbuf.at[pl.ds(pos[c], LANES)], pk, mask=m)
            new.append(pos[c] + jnp.sum(m.astype(jnp.int32)))
        return tuple(new)

    lax.fori_loop(0, SHARD // LANES, part_body, pos0)
    mvbuf[pl.ds(0, LANES)] = pref
    mvbuf[pl.ds(LANES, LANES)] = cntv


def _sc_layer_body(yp_hbm, stream_hbm, meta_hbm, agg_hbm, ypk, aslab, ebuf0,
                   ebuf1, mall, sem0, sem1):
    wid = _wid()
    pltpu.sync_copy(meta_hbm, mall)
    pltpu.sync_copy(yp_hbm.at[pl.ds(wid * 2 * NP, 2 * NP)], ypk)
    _zero_ref(aslab, FPT * NP)
    himask = jnp.full((LANES,), HIMASK, jnp.int32)
    ebufs = (ebuf0, ebuf1)
    sems = (sem0, sem1)

    pltpu.async_copy(stream_hbm.at[pl.ds(0, RCAP)], ebuf0, sem0)

    def process(w2, ebuf):
        lstart = mall[pl.ds(w2 * 2 * LANES, LANES)]
        cntv = mall[pl.ds(w2 * 2 * LANES + LANES, LANES)]
        maxc = jnp.max(cntv)

        @plsc.parallel_loop(0, maxc, unroll=UNROLL)
        def hot(j):
            valid = j < cntv
            pk = plsc.load_gather(ebuf, (lstart + j,), mask=valid)
            s = pk & 0x3FFF
            d = lax.shift_right_logical(pk, 14)
            for p in range(FPT // 2):
                w = plsc.load_gather(ypk, (s + p * NP,), mask=valid)
                lof = plsc.bitcast(lax.shift_left(w, 16), jnp.float32)
                hif = plsc.bitcast(w & himask, jnp.float32)
                plsc.addupdate_scatter(aslab, (d + (2 * p) * NP,), lof,
                                       mask=valid)
                plsc.addupdate_scatter(aslab, (d + (2 * p + 1) * NP,), hif,
                                       mask=valid)

    def pair(pi, _):
        for sl in range(2):
            w2 = pi * 2 + sl
            nxt = w2 + 1

            @pl.when(nxt < NW)
            def _():
                pltpu.async_copy(stream_hbm.at[pl.ds(nxt * RCAP, RCAP)],
                                 ebufs[1 - sl], sems[1 - sl])

            pltpu.make_async_copy(stream_hbm.at[pl.ds(0, RCAP)], ebufs[sl],
                                  sems[sl]).wait()
            process(w2, ebufs[sl])
        return 0

    lax.fori_loop(0, NW // 2, pair, 0)
    pltpu.sync_copy(aslab, agg_hbm.at[pl.ds(wid * FPT * NP, FPT * NP)])


def _sc_layer(yp_flat, stream, meta):
    mesh = plsc.VectorSubcoreMesh(core_axis_name="c", subcore_axis_name="s")
    k = pl.kernel(
        _sc_layer_body,
        out_type=jax.ShapeDtypeStruct((D * NP,), jnp.float32),
        mesh=mesh,
        compiler_params=pltpu.CompilerParams(needs_layout_passes=False),
        scratch_types=[
            pltpu.VMEM((2 * NP,), jnp.int32),
            pltpu.VMEM((FPT * NP,), jnp.float32),
            pltpu.VMEM((RCAP,), jnp.int32),
            pltpu.VMEM((RCAP,), jnp.int32),
            pltpu.VMEM((NW * 2 * LANES,), jnp.int32),
            pltpu.SemaphoreType.DMA,
            pltpu.SemaphoreType.DMA,
        ],
    )
    return k(yp_flat, stream, meta)


def _pack_pairs(y):
    au = lax.bitcast_convert_type(y[0:64, :], jnp.int32)
    bu = lax.bitcast_convert_type(y[64:128, :], jnp.int32)
    rnd = jnp.int32(0x8000)
    lo = lax.shift_right_logical(au + rnd, 16)
    hi = (bu + rnd) & jnp.int32(HIMASK)
    return lo | hi


def _tc1_body(h_ref, w1_ref, degp_ref, y_ref, on_ref, in_ref):
    deg = jnp.sum(degp_ref[...], axis=1)
    onorm = lax.rsqrt(jnp.maximum(deg[0:1, :], 1.0))
    inorm = lax.rsqrt(jnp.maximum(deg[1:2, :], 1.0))
    y = lax.dot_general(w1_ref[...], h_ref[...], (((0,), (1,)), ((), ())),
                        preferred_element_type=jnp.float32)
    y_ref[...] = _pack_pairs(y * onorm)
    on_ref[...] = onorm
    in_ref[...] = inorm


def _tc1(h_pad, w1p, degp):
    return pl.pallas_call(
        _tc1_body,
        grid=(GRID,),
        in_specs=[
            pl.BlockSpec((BN, D), lambda i: (i, 0)),
            pl.BlockSpec((D, D), lambda i: (0, 0)),
            pl.BlockSpec((2, NW, BN), lambda i: (0, 0, i)),
        ],
        out_specs=[
            pl.BlockSpec((D // 2, BN), lambda i: (0, i)),
            pl.BlockSpec((1, BN), lambda i: (0, i)),
            pl.BlockSpec((1, BN), lambda i: (0, i)),
        ],
        out_shape=[
            jax.ShapeDtypeStruct((D // 2, NP), jnp.int32),
            jax.ShapeDtypeStruct((1, NP), jnp.float32),
            jax.ShapeDtypeStruct((1, NP), jnp.float32),
        ],
    )(h_pad, w1p, degp)


def _tc_mid_body(agg_ref, in_ref, on_ref, b_ref, w_ref, y_ref):
    hprev = jnp.maximum(agg_ref[...] * in_ref[...] + b_ref[...], 0.0)
    y = lax.dot_general(w_ref[...], hprev, (((0,), (0,)), ((), ())),
                        preferred_element_type=jnp.float32)
    y_ref[...] = _pack_pairs(y * on_ref[...])


def _tc_mid(agg, inorm, onorm, b_prev, wp):
    return pl.pallas_call(
        _tc_mid_body,
        grid=(GRID,),
        in_specs=[
            pl.BlockSpec((D, BN), lambda i: (0, i)),
            pl.BlockSpec((1, BN), lambda i: (0, i)),
            pl.BlockSpec((1, BN), lambda i: (0, i)),
            pl.BlockSpec((D, 1), lambda i: (0, 0)),
            pl.BlockSpec((D, D), lambda i: (0, 0)),
        ],
        out_specs=pl.BlockSpec((D // 2, BN), lambda i: (0, i)),
        out_shape=jax.ShapeDtypeStruct((D // 2, NP), jnp.int32),
    )(agg, inorm, onorm, b_prev, wp)


def _tc_fin_body(agg_ref, in_ref, b_ref, o_ref):
    i = pl.program_id(0)
    h3 = jnp.maximum(agg_ref[...] * in_ref[...] + b_ref[...], 0.0)
    col = lax.broadcasted_iota(jnp.int32, (1, BN), 1) + i * BN
    h3 = jnp.where(col < N, h3, 0.0)
    part = jnp.sum(h3, axis=1)

    @pl.when(i == 0)
    def _():
        o_ref[...] = jnp.zeros_like(o_ref)

    o_ref[...] += part[None, :]

    @pl.when(i == GRID - 1)
    def _():
        o_ref[...] *= jnp.float32(1.0 / N)


def _tc_fin(agg3, inorm, b3):
    return pl.pallas_call(
        _tc_fin_body,
        grid=(GRID,),
        in_specs=[
            pl.BlockSpec((D, BN), lambda i: (0, i)),
            pl.BlockSpec((1, BN), lambda i: (0, i)),
            pl.BlockSpec((D, 1), lambda i: (0, 0)),
        ],
        out_specs=pl.BlockSpec((1, D), lambda i: (0, 0)),
        out_shape=jax.ShapeDtypeStruct((1, D), jnp.float32),
    )(agg3, inorm, b3)


_PERM = np.concatenate([np.arange(0, D, 2), np.arange(1, D, 2)])


def kernel(h, edge_index, W1, b1, W2, b2, W3, b3):
    src = edge_index[0]
    dst = edge_index[1]
    h_pad = jnp.pad(h, ((0, NP - N), (0, 0)))
    perm = jnp.asarray(_PERM)
    w1p = W1[:, perm]
    w2p = W2[:, perm]
    w3p = W3[:, perm]

    degp_flat, stream, meta = _sc_deg(src, dst)
    degp = degp_flat.reshape(2, NW, NP)
    y1p, onorm, inorm = _tc1(h_pad, w1p, degp)

    agg1 = _sc_layer(y1p.reshape(-1), stream, meta).reshape(D, NP)
    y2p = _tc_mid(agg1, inorm, onorm, b1.reshape(D, 1), w2p)

    agg2 = _sc_layer(y2p.reshape(-1), stream, meta).reshape(D, NP)
    y3p = _tc_mid(agg2, inorm, onorm, b2.reshape(D, 1), w3p)

    agg3 = _sc_layer(y3p.reshape(-1), stream, meta).reshape(D, NP)
    return _tc_fin(agg3, inorm, b3.reshape(D, 1))
